# Initial kernel scaffold; baseline (speedup 1.0000x reference)
import jax
import jax.numpy as jnp
from jax.experimental import pallas as pl


def kernel(hidden_states, gate_w, gate_b):
    raise NotImplementedError("write your pallas kernel here")


# speedup vs baseline: 1.0897x; 1.0897x over previous
import functools

import jax
import jax.numpy as jnp
from jax.experimental import pallas as pl
from jax.experimental.pallas import tpu as pltpu

_TOP_K = 8


def _router_body(x_ref, w_ref, b_ref, logits_ref, weights_ref, experts_ref):
    x = x_ref[...]
    w = w_ref[...]
    logits = jnp.dot(x, w, preferred_element_type=jnp.float32) + b_ref[...]
    logits_ref[...] = logits

    num_experts = logits.shape[-1]
    iota = jax.lax.broadcasted_iota(jnp.int32, logits.shape, 1)
    work = logits
    vals = []
    idxs = []
    for _ in range(_TOP_K):
        m = jnp.max(work, axis=1, keepdims=True)
        idx = jnp.min(jnp.where(work == m, iota, num_experts),
                      axis=1, keepdims=True)
        vals.append(m)
        idxs.append(idx)
        work = jnp.where(iota == idx, -jnp.inf, work)
    topv = jnp.concatenate(vals, axis=1)
    topi = jnp.concatenate(idxs, axis=1)

    e = jnp.exp(topv - topv[:, :1])
    weights_ref[...] = e / jnp.sum(e, axis=1, keepdims=True)
    experts_ref[...] = topi


@jax.jit
def kernel(hidden_states, gate_w, gate_b):
    num_tokens, hidden_dim = hidden_states.shape
    num_experts = gate_w.shape[0]
    w_t = gate_w.T
    bias = gate_b.reshape(1, num_experts)

    tb = 1024
    while num_tokens % tb:
        tb //= 2
    grid = (num_tokens // tb,)

    logits, weights, experts = pl.pallas_call(
        _router_body,
        grid=grid,
        in_specs=[
            pl.BlockSpec((tb, hidden_dim), lambda i: (i, 0)),
            pl.BlockSpec((hidden_dim, num_experts), lambda i: (0, 0)),
            pl.BlockSpec((1, num_experts), lambda i: (0, 0)),
        ],
        out_specs=[
            pl.BlockSpec((tb, num_experts), lambda i: (i, 0)),
            pl.BlockSpec((tb, _TOP_K), lambda i: (i, 0)),
            pl.BlockSpec((tb, _TOP_K), lambda i: (i, 0)),
        ],
        out_shape=[
            jax.ShapeDtypeStruct((num_tokens, num_experts), jnp.float32),
            jax.ShapeDtypeStruct((num_tokens, _TOP_K), jnp.float32),
            jax.ShapeDtypeStruct((num_tokens, _TOP_K), jnp.int32),
        ],
        compiler_params=pltpu.CompilerParams(
            dimension_semantics=("arbitrary",),
        ),
    )(hidden_states, w_t, bias)
    return (logits, weights, experts)
